# baseline (speedup 1.0000x reference)
import functools

import jax
import jax.numpy as jnp
from jax import lax
from jax.experimental import pallas as pl
from jax.experimental.pallas import tpu as pltpu
from jax.experimental.pallas import tpu_sc as plsc

NC = 2
NS = 16
NW = NC * NS
DW = 16


def _mesh():
    return plsc.VectorSubcoreMesh(core_axis_name="c", subcore_axis_name="s")


_SC_PARAMS = pltpu.CompilerParams(use_tc_tiling_on_sc=False)


def _sc_degree(dst3, ones, zeros):
    _, NCHUNK, K = dst3.shape
    NP = zeros.shape[0]
    RPT = NP // NS
    NB = 10
    assert NCHUNK % NB == 0

    @functools.partial(
        pl.kernel,
        out_type=jax.ShapeDtypeStruct((NC, NP, DW), jnp.float32),
        mesh=_mesh(),
        compiler_params=_SC_PARAMS,
        scratch_types=[
            pltpu.VMEM((NCHUNK, K), jnp.int32),
            pltpu.VMEM((K, DW), jnp.float32),
            pltpu.VMEM_SHARED((NP, DW), jnp.float32),
            pltpu.SemaphoreType.DMA,
        ],
    )
    def degk(dst_hbm, ones_hbm, zeros_hbm, out_hbm, dst_v, ones_v, acc, sem):
        c = lax.axis_index("c")
        s = lax.axis_index("s")
        w = c * NS + s
        pltpu.sync_copy(dst_hbm.at[w], dst_v)
        pltpu.sync_copy(ones_hbm, ones_v)
        pltpu.sync_copy(zeros_hbm.at[pl.ds(s * RPT, RPT)],
                        acc.at[pl.ds(s * RPT, RPT)])
        plsc.subcore_barrier()

        def outer(i, carry):
            for b in range(NB):
                pltpu.sync_copy(ones_v, acc.at[dst_v.at[i * NB + b]], add=True)
            return carry

        lax.fori_loop(0, NCHUNK // NB, outer, 0)
        plsc.subcore_barrier()
        pltpu.sync_copy(acc.at[pl.ds(s * RPT, RPT)],
                        out_hbm.at[c, pl.ds(s * RPT, RPT)])

    return degk(dst3, ones, zeros)


def _sc_aggregate(feat, src3, dst3, zeros, stage=False):
    N, PD = feat.shape
    NP = zeros.shape[0]
    _, NCHUNK, K = src3.shape
    RPT = NP // NS
    RPN = N // NS
    assert NCHUNK % 2 == 0 and NCHUNK >= 4

    scratch = [
        pltpu.VMEM((NCHUNK, K), jnp.int32),
        pltpu.VMEM((NCHUNK, K), jnp.int32),
        pltpu.VMEM((K, PD), jnp.float32),
        pltpu.VMEM((K, PD), jnp.float32),
        pltpu.VMEM_SHARED((NP, PD), jnp.float32),
        pltpu.SemaphoreType.DMA,
        pltpu.SemaphoreType.DMA,
    ]
    if stage:
        scratch.append(pltpu.VMEM_SHARED((N, PD), jnp.float32))

    @functools.partial(
        pl.kernel,
        out_type=jax.ShapeDtypeStruct((NC, NP, PD), jnp.float32),
        mesh=_mesh(),
        compiler_params=_SC_PARAMS,
        scratch_types=scratch,
    )
    def aggk(feat_hbm, src_hbm, dst_hbm, zeros_hbm, out_hbm,
             src_v, dst_v, buf0, buf1, acc, sem0, sem1, *rest):
        c = lax.axis_index("c")
        s = lax.axis_index("s")
        w = c * NS + s
        rows = pl.ds(s * RPT, RPT)
        pltpu.sync_copy(src_hbm.at[w], src_v)
        pltpu.sync_copy(dst_hbm.at[w], dst_v)
        pltpu.sync_copy(zeros_hbm.at[rows], acc.at[rows])
        if stage:
            feat_src = rest[0]
            own = pl.ds(s * RPN, RPN)
            pltpu.sync_copy(feat_hbm.at[own], feat_src.at[own])
        else:
            feat_src = feat_hbm
        plsc.subcore_barrier()

        pltpu.async_copy(feat_src.at[src_v.at[0]], buf0, sem0)
        pltpu.async_copy(feat_src.at[src_v.at[1]], buf1, sem1)

        def body(i, carry):
            j = 2 * i
            pltpu.make_async_copy(feat_src.at[src_v.at[0]], buf0, sem0).wait()
            pltpu.sync_copy(buf0, acc.at[dst_v.at[j]], add=True)
            pltpu.async_copy(feat_src.at[src_v.at[j + 2]], buf0, sem0)
            pltpu.make_async_copy(feat_src.at[src_v.at[1]], buf1, sem1).wait()
            pltpu.sync_copy(buf1, acc.at[dst_v.at[j + 1]], add=True)
            pltpu.async_copy(feat_src.at[src_v.at[j + 3]], buf1, sem1)
            return carry

        lax.fori_loop(0, NCHUNK // 2 - 1, body, 0)
        pltpu.make_async_copy(feat_src.at[src_v.at[0]], buf0, sem0).wait()
        pltpu.sync_copy(buf0, acc.at[dst_v.at[NCHUNK - 2]], add=True)
        pltpu.make_async_copy(feat_src.at[src_v.at[1]], buf1, sem1).wait()
        pltpu.sync_copy(buf1, acc.at[dst_v.at[NCHUNK - 1]], add=True)

        plsc.subcore_barrier()
        pltpu.sync_copy(acc.at[rows], out_hbm.at[c, rows])

    return aggk(feat, src3, dst3, zeros)


def _tc_matmul1(x, W1, block):
    N, Din = x.shape
    Dh = W1.shape[1]
    grid = N // block

    def body(x_ref, w_ref, h_ref):
        h_ref[...] = jnp.dot(x_ref[...], w_ref[...],
                             preferred_element_type=jnp.float32)

    return pl.pallas_call(
        body,
        grid=(grid,),
        in_specs=[
            pl.BlockSpec((block, Din), lambda i: (i, 0)),
            pl.BlockSpec((Din, Dh), lambda i: (0, 0)),
        ],
        out_specs=pl.BlockSpec((block, Dh), lambda i: (i, 0)),
        out_shape=jax.ShapeDtypeStruct((N, Dh), jnp.float32),
    )(x, W1)


def _tc_layer1(H, b1, degp, block):
    N, Dh = H.shape
    grid = N // block

    def body(h_ref, b_ref, deg_ref, hs_ref, s1_ref, dinv_ref):
        d = deg_ref[...]
        deg = d[0, :, 0:1] + d[1, :, 0:1] + 1.0
        dinv = lax.rsqrt(deg)
        H = h_ref[...]
        hs_ref[...] = H * dinv
        s1_ref[...] = b_ref[...] + H * (dinv * dinv)
        dinv_ref[...] = dinv

    return pl.pallas_call(
        body,
        grid=(grid,),
        in_specs=[
            pl.BlockSpec((block, Dh), lambda i: (i, 0)),
            pl.BlockSpec((1, Dh), lambda i: (0, 0)),
            pl.BlockSpec((NC, block, DW), lambda i: (0, i, 0)),
        ],
        out_specs=[
            pl.BlockSpec((block, Dh), lambda i: (i, 0)),
            pl.BlockSpec((block, Dh), lambda i: (i, 0)),
            pl.BlockSpec((block, 1), lambda i: (i, 0)),
        ],
        out_shape=[
            jax.ShapeDtypeStruct((N, Dh), jnp.float32),
            jax.ShapeDtypeStruct((N, Dh), jnp.float32),
            jax.ShapeDtypeStruct((N, 1), jnp.float32),
        ],
    )(H, b1, degp)


def _tc_layer2(U, dinv, S1, W2, b2, block):
    N, Dh = S1.shape
    Do = W2.shape[1]
    grid = N // block

    def body(u_ref, dinv_ref, s1_ref, w_ref, b_ref, gs_ref, s2_ref):
        u = u_ref[...]
        dinv = dinv_ref[...]
        h = jnp.maximum((u[0] + u[1]) * dinv + s1_ref[...], 0.0)
        G = jnp.dot(h, w_ref[...], preferred_element_type=jnp.float32)
        gs_ref[...] = G * dinv
        s2_ref[...] = b_ref[...] + G * (dinv * dinv)

    return pl.pallas_call(
        body,
        grid=(grid,),
        in_specs=[
            pl.BlockSpec((NC, block, Dh), lambda i: (0, i, 0)),
            pl.BlockSpec((block, 1), lambda i: (i, 0)),
            pl.BlockSpec((block, Dh), lambda i: (i, 0)),
            pl.BlockSpec((Dh, Do), lambda i: (0, 0)),
            pl.BlockSpec((1, Do), lambda i: (0, 0)),
        ],
        out_specs=[
            pl.BlockSpec((block, Do), lambda i: (i, 0)),
            pl.BlockSpec((block, Do), lambda i: (i, 0)),
        ],
        out_shape=[
            jax.ShapeDtypeStruct((N, Do), jnp.float32),
            jax.ShapeDtypeStruct((N, Do), jnp.float32),
        ],
    )(U, dinv, S1, W2, b2)


def _tc_finish(V, dinv, S2, block):
    N, Do = S2.shape
    grid = N // block

    def body(v_ref, dinv_ref, s2_ref, z_ref):
        v = v_ref[...]
        z_ref[...] = (v[0] + v[1]) * dinv_ref[...] + s2_ref[...]

    return pl.pallas_call(
        body,
        grid=(grid,),
        in_specs=[
            pl.BlockSpec((NC, block, Do), lambda i: (0, i, 0)),
            pl.BlockSpec((block, 1), lambda i: (i, 0)),
            pl.BlockSpec((block, Do), lambda i: (i, 0)),
        ],
        out_specs=pl.BlockSpec((block, Do), lambda i: (i, 0)),
        out_shape=jax.ShapeDtypeStruct((N, Do), jnp.float32),
    )(V, dinv, S2)


def kernel(x, edge_index, W1, b1, W2, b2, W3, b3):
    del W3, b3
    N, Din = x.shape
    E = edge_index.shape[1]
    Dh = W1.shape[1]
    Do = W2.shape[1]
    assert E % NW == 0 and N % NS == 0
    EPW = E // NW
    K = 100
    assert EPW % K == 0
    NCHUNK = EPW // K

    ei = edge_index.astype(jnp.int32)
    src3 = ei[0].reshape(NW, NCHUNK, K)
    dst3 = ei[1].reshape(NW, NCHUNK, K)
    NP = ((N + 8 * NS - 1) // (8 * NS)) * (8 * NS)
    ones = jnp.ones((K, DW), jnp.float32)
    zeros_h = jnp.zeros((NP, Dh), jnp.float32)
    zeros_o = jnp.zeros((NP, Do), jnp.float32)
    zeros_d = jnp.zeros((NP, DW), jnp.float32)

    block = 5000
    degp = _sc_degree(dst3, ones, zeros_d)
    H = _tc_matmul1(x, W1, block)
    Hs, S1, dinv = _tc_layer1(H, b1.reshape(1, Dh), degp, block)
    U = _sc_aggregate(Hs, src3, dst3, zeros_h)
    Gs, S2 = _tc_layer2(U, dinv, S1, W2, b2.reshape(1, Do), block)
    V = _sc_aggregate(Gs, src3, dst3, zeros_o, stage=True)
    return _tc_finish(V, dinv, S2, block)


# speedup vs baseline: 1.0044x; 1.0044x over previous
import functools

import jax
import jax.numpy as jnp
from jax import lax
from jax.experimental import pallas as pl
from jax.experimental.pallas import tpu as pltpu
from jax.experimental.pallas import tpu_sc as plsc

NC = 2
NS = 16
NW = NC * NS
DW = 16


def _mesh():
    return plsc.VectorSubcoreMesh(core_axis_name="c", subcore_axis_name="s")


_SC_PARAMS = pltpu.CompilerParams(use_tc_tiling_on_sc=False)


def _sc_degree(dst3, ones, zeros):
    _, NCHUNK, K = dst3.shape
    NP = zeros.shape[0]
    RPT = NP // NS
    NB = 10
    assert NCHUNK % NB == 0

    @functools.partial(
        pl.kernel,
        out_type=jax.ShapeDtypeStruct((NC, NP, DW), jnp.float32),
        mesh=_mesh(),
        compiler_params=_SC_PARAMS,
        scratch_types=[
            pltpu.VMEM((NCHUNK, K), jnp.int32),
            pltpu.VMEM((K, DW), jnp.float32),
            pltpu.VMEM_SHARED((NP, DW), jnp.float32),
            pltpu.SemaphoreType.DMA,
        ],
    )
    def degk(dst_hbm, ones_hbm, zeros_hbm, out_hbm, dst_v, ones_v, acc, sem):
        c = lax.axis_index("c")
        s = lax.axis_index("s")
        w = c * NS + s
        pltpu.sync_copy(dst_hbm.at[w], dst_v)
        pltpu.sync_copy(ones_hbm, ones_v)
        pltpu.sync_copy(zeros_hbm.at[pl.ds(s * RPT, RPT)],
                        acc.at[pl.ds(s * RPT, RPT)])
        plsc.subcore_barrier()

        def outer(i, carry):
            for b in range(NB):
                pltpu.sync_copy(ones_v, acc.at[dst_v.at[i * NB + b]], add=True)
            return carry

        lax.fori_loop(0, NCHUNK // NB, outer, 0)
        plsc.subcore_barrier()
        pltpu.sync_copy(acc.at[pl.ds(s * RPT, RPT)],
                        out_hbm.at[c, pl.ds(s * RPT, RPT)])

    return degk(dst3, ones, zeros)


def _sc_aggregate(feat, src3, dst3, zeros, stage=False):
    N, PD = feat.shape
    NP = zeros.shape[0]
    _, NCHUNK, K = src3.shape
    RPT = NP // NS
    RPN = N // NS
    assert NCHUNK % 2 == 0 and NCHUNK >= 4

    scratch = [
        pltpu.VMEM((NCHUNK, K), jnp.int32),
        pltpu.VMEM((NCHUNK, K), jnp.int32),
        pltpu.VMEM((K, PD), jnp.float32),
        pltpu.VMEM((K, PD), jnp.float32),
        pltpu.VMEM_SHARED((NP, PD), jnp.float32),
        pltpu.SemaphoreType.DMA,
        pltpu.SemaphoreType.DMA,
    ]
    if stage:
        scratch.append(pltpu.VMEM_SHARED((N, PD), jnp.float32))

    @functools.partial(
        pl.kernel,
        out_type=jax.ShapeDtypeStruct((NC, NP, PD), jnp.float32),
        mesh=_mesh(),
        compiler_params=_SC_PARAMS,
        scratch_types=scratch,
    )
    def aggk(feat_hbm, src_hbm, dst_hbm, zeros_hbm, out_hbm,
             src_v, dst_v, buf0, buf1, acc, sem0, sem1, *rest):
        c = lax.axis_index("c")
        s = lax.axis_index("s")
        w = c * NS + s
        rows = pl.ds(s * RPT, RPT)
        pltpu.sync_copy(src_hbm.at[w], src_v)
        pltpu.sync_copy(dst_hbm.at[w], dst_v)
        pltpu.sync_copy(zeros_hbm.at[rows], acc.at[rows])
        if stage:
            feat_src = rest[0]
            own = pl.ds(s * RPN, RPN)
            pltpu.sync_copy(feat_hbm.at[own], feat_src.at[own])
        else:
            feat_src = feat_hbm
        plsc.subcore_barrier()

        pltpu.async_copy(feat_src.at[src_v.at[0]], buf0, sem0)
        pltpu.async_copy(feat_src.at[src_v.at[1]], buf1, sem1)

        def body(i, carry):
            j = 2 * i
            pltpu.make_async_copy(feat_src.at[src_v.at[0]], buf0, sem0).wait()
            pltpu.sync_copy(buf0, acc.at[dst_v.at[j]], add=True)
            pltpu.async_copy(feat_src.at[src_v.at[j + 2]], buf0, sem0)
            pltpu.make_async_copy(feat_src.at[src_v.at[1]], buf1, sem1).wait()
            pltpu.sync_copy(buf1, acc.at[dst_v.at[j + 1]], add=True)
            pltpu.async_copy(feat_src.at[src_v.at[j + 3]], buf1, sem1)
            return carry

        lax.fori_loop(0, NCHUNK // 2 - 1, body, 0)
        pltpu.make_async_copy(feat_src.at[src_v.at[0]], buf0, sem0).wait()
        pltpu.sync_copy(buf0, acc.at[dst_v.at[NCHUNK - 2]], add=True)
        pltpu.make_async_copy(feat_src.at[src_v.at[1]], buf1, sem1).wait()
        pltpu.sync_copy(buf1, acc.at[dst_v.at[NCHUNK - 1]], add=True)

        plsc.subcore_barrier()
        pltpu.sync_copy(acc.at[rows], out_hbm.at[c, rows])

    return aggk(feat, src3, dst3, zeros)


def _tc_matmul1(x, W1, block):
    N, Din = x.shape
    Dh = W1.shape[1]
    grid = N // block

    def body(x_ref, w_ref, h_ref):
        h_ref[...] = jnp.dot(x_ref[...], w_ref[...],
                             preferred_element_type=jnp.float32)

    return pl.pallas_call(
        body,
        grid=(grid,),
        in_specs=[
            pl.BlockSpec((block, Din), lambda i: (i, 0)),
            pl.BlockSpec((Din, Dh), lambda i: (0, 0)),
        ],
        out_specs=pl.BlockSpec((block, Dh), lambda i: (i, 0)),
        out_shape=jax.ShapeDtypeStruct((N, Dh), jnp.float32),
    )(x, W1)


def _tc_layer1(H, b1, degp, block):
    N, Dh = H.shape
    grid = N // block

    def body(h_ref, b_ref, deg_ref, hs_ref, s1_ref, dinv_ref):
        d = deg_ref[...]
        deg = d[0, :, 0:1] + d[1, :, 0:1] + 1.0
        dinv = lax.rsqrt(deg)
        H = h_ref[...]
        hs_ref[...] = H * dinv
        s1_ref[...] = b_ref[...] + H * (dinv * dinv)
        dinv_ref[...] = dinv

    return pl.pallas_call(
        body,
        grid=(grid,),
        in_specs=[
            pl.BlockSpec((block, Dh), lambda i: (i, 0)),
            pl.BlockSpec((1, Dh), lambda i: (0, 0)),
            pl.BlockSpec((NC, block, DW), lambda i: (0, i, 0)),
        ],
        out_specs=[
            pl.BlockSpec((block, Dh), lambda i: (i, 0)),
            pl.BlockSpec((block, Dh), lambda i: (i, 0)),
            pl.BlockSpec((block, 1), lambda i: (i, 0)),
        ],
        out_shape=[
            jax.ShapeDtypeStruct((N, Dh), jnp.float32),
            jax.ShapeDtypeStruct((N, Dh), jnp.float32),
            jax.ShapeDtypeStruct((N, 1), jnp.float32),
        ],
    )(H, b1, degp)


def _tc_layer2(U, dinv, S1, W2, b2, block):
    N, Dh = S1.shape
    Do = W2.shape[1]
    grid = N // block

    def body(u_ref, dinv_ref, s1_ref, w_ref, b_ref, gs_ref, s2_ref):
        u = u_ref[...]
        dinv = dinv_ref[...]
        h = jnp.maximum((u[0] + u[1]) * dinv + s1_ref[...], 0.0)
        G = jnp.dot(h, w_ref[...], preferred_element_type=jnp.float32)
        gs_ref[...] = G * dinv
        s2_ref[...] = b_ref[...] + G * (dinv * dinv)

    return pl.pallas_call(
        body,
        grid=(grid,),
        in_specs=[
            pl.BlockSpec((NC, block, Dh), lambda i: (0, i, 0)),
            pl.BlockSpec((block, 1), lambda i: (i, 0)),
            pl.BlockSpec((block, Dh), lambda i: (i, 0)),
            pl.BlockSpec((Dh, Do), lambda i: (0, 0)),
            pl.BlockSpec((1, Do), lambda i: (0, 0)),
        ],
        out_specs=[
            pl.BlockSpec((block, Do), lambda i: (i, 0)),
            pl.BlockSpec((block, Do), lambda i: (i, 0)),
        ],
        out_shape=[
            jax.ShapeDtypeStruct((N, Do), jnp.float32),
            jax.ShapeDtypeStruct((N, Do), jnp.float32),
        ],
    )(U, dinv, S1, W2, b2)


def _tc_finish(V, dinv, S2, block):
    N, Do = S2.shape
    grid = N // block

    def body(v_ref, dinv_ref, s2_ref, z_ref):
        v = v_ref[...]
        z_ref[...] = (v[0] + v[1]) * dinv_ref[...] + s2_ref[...]

    return pl.pallas_call(
        body,
        grid=(grid,),
        in_specs=[
            pl.BlockSpec((NC, block, Do), lambda i: (0, i, 0)),
            pl.BlockSpec((block, 1), lambda i: (i, 0)),
            pl.BlockSpec((block, Do), lambda i: (i, 0)),
        ],
        out_specs=pl.BlockSpec((block, Do), lambda i: (i, 0)),
        out_shape=jax.ShapeDtypeStruct((N, Do), jnp.float32),
    )(V, dinv, S2)


def kernel(x, edge_index, W1, b1, W2, b2, W3, b3):
    del W3, b3
    N, Din = x.shape
    E = edge_index.shape[1]
    Dh = W1.shape[1]
    Do = W2.shape[1]
    assert E % NW == 0 and N % NS == 0
    EPW = E // NW
    K = 100
    assert EPW % K == 0
    NCHUNK = EPW // K

    ei = edge_index.astype(jnp.int32)
    src3 = ei[0].reshape(NW, NCHUNK, K)
    dst3 = ei[1].reshape(NW, NCHUNK, K)
    NP = ((N + 8 * NS - 1) // (8 * NS)) * (8 * NS)
    ones = jnp.ones((K, DW), jnp.float32)
    zeros_h = jnp.zeros((NP, Dh), jnp.float32)
    zeros_o = jnp.zeros((NP, Do), jnp.float32)
    zeros_d = jnp.zeros((NP, DW), jnp.float32)

    block = 10000
    degp = _sc_degree(dst3, ones, zeros_d)
    H = _tc_matmul1(x, W1, block)
    Hs, S1, dinv = _tc_layer1(H, b1.reshape(1, Dh), degp, block)
    U = _sc_aggregate(Hs, src3, dst3, zeros_h)
    Gs, S2 = _tc_layer2(U, dinv, S1, W2, b2.reshape(1, Do), block)
    V = _sc_aggregate(Gs, src3, dst3, zeros_o)
    return _tc_finish(V, dinv, S2, block)


# speedup vs baseline: 1.0182x; 1.0137x over previous
import functools

import jax
import jax.numpy as jnp
from jax import lax
from jax.experimental import pallas as pl
from jax.experimental.pallas import tpu as pltpu
from jax.experimental.pallas import tpu_sc as plsc

NC = 2
NS = 16
NW = NC * NS
DW = 16


def _mesh():
    return plsc.VectorSubcoreMesh(core_axis_name="c", subcore_axis_name="s")


_SC_PARAMS = pltpu.CompilerParams(use_tc_tiling_on_sc=False)


def _sc_degree(dst3, ones, zeros):
    _, NCHUNK, K = dst3.shape
    NP = zeros.shape[0]
    RPT = NP // NS
    NB = 10
    assert NCHUNK % NB == 0

    @functools.partial(
        pl.kernel,
        out_type=jax.ShapeDtypeStruct((NC, NP, DW), jnp.float32),
        mesh=_mesh(),
        compiler_params=_SC_PARAMS,
        scratch_types=[
            pltpu.VMEM((NCHUNK, K), jnp.int32),
            pltpu.VMEM((K, DW), jnp.float32),
            pltpu.VMEM_SHARED((NP, DW), jnp.float32),
            pltpu.SemaphoreType.DMA,
        ],
    )
    def degk(dst_hbm, ones_hbm, zeros_hbm, out_hbm, dst_v, ones_v, acc, sem):
        c = lax.axis_index("c")
        s = lax.axis_index("s")
        w = c * NS + s
        pltpu.sync_copy(dst_hbm.at[w], dst_v)
        pltpu.sync_copy(ones_hbm, ones_v)
        pltpu.sync_copy(zeros_hbm.at[pl.ds(s * RPT, RPT)],
                        acc.at[pl.ds(s * RPT, RPT)])
        plsc.subcore_barrier()

        def outer(i, carry):
            for b in range(NB):
                pltpu.sync_copy(ones_v, acc.at[dst_v.at[i * NB + b]], add=True)
            return carry

        lax.fori_loop(0, NCHUNK // NB, outer, 0)
        plsc.subcore_barrier()
        pltpu.sync_copy(acc.at[pl.ds(s * RPT, RPT)],
                        out_hbm.at[c, pl.ds(s * RPT, RPT)])

    return degk(dst3, ones, zeros)


def _sc_aggregate(feat, src3, dst3, zeros):
    N, PD = feat.shape
    NP = zeros.shape[0]
    _, NCHUNK, K = src3.shape
    RPT = NP // NS
    assert NCHUNK % 2 == 0 and NCHUNK >= 4

    @functools.partial(
        pl.kernel,
        out_type=jax.ShapeDtypeStruct((NC, NP, PD), jnp.float32),
        mesh=_mesh(),
        compiler_params=_SC_PARAMS,
        scratch_types=[
            pltpu.VMEM((NCHUNK, K), jnp.int32),
            pltpu.VMEM((NCHUNK, K), jnp.int32),
            pltpu.VMEM((K, PD), jnp.float32),
            pltpu.VMEM((K, PD), jnp.float32),
            pltpu.VMEM_SHARED((NP, PD), jnp.float32),
            pltpu.SemaphoreType.DMA,
            pltpu.SemaphoreType.DMA,
        ],
    )
    def aggk(feat_hbm, src_hbm, dst_hbm, zeros_hbm, out_hbm,
             src_v, dst_v, buf0, buf1, acc, sem0, sem1):
        c = lax.axis_index("c")
        s = lax.axis_index("s")
        w = c * NS + s
        rows = pl.ds(s * RPT, RPT)
        pltpu.sync_copy(src_hbm.at[w], src_v)
        pltpu.sync_copy(dst_hbm.at[w], dst_v)
        pltpu.sync_copy(zeros_hbm.at[rows], acc.at[rows])
        feat_src = feat_hbm
        plsc.subcore_barrier()

        pltpu.async_copy(feat_src.at[src_v.at[0]], buf0, sem0)
        pltpu.async_copy(feat_src.at[src_v.at[1]], buf1, sem1)

        def body(i, carry):
            j = 2 * i
            pltpu.make_async_copy(feat_src.at[src_v.at[0]], buf0, sem0).wait()
            pltpu.sync_copy(buf0, acc.at[dst_v.at[j]], add=True)
            pltpu.async_copy(feat_src.at[src_v.at[j + 2]], buf0, sem0)
            pltpu.make_async_copy(feat_src.at[src_v.at[1]], buf1, sem1).wait()
            pltpu.sync_copy(buf1, acc.at[dst_v.at[j + 1]], add=True)
            pltpu.async_copy(feat_src.at[src_v.at[j + 3]], buf1, sem1)
            return carry

        lax.fori_loop(0, NCHUNK // 2 - 1, body, 0)
        pltpu.make_async_copy(feat_src.at[src_v.at[0]], buf0, sem0).wait()
        pltpu.sync_copy(buf0, acc.at[dst_v.at[NCHUNK - 2]], add=True)
        pltpu.make_async_copy(feat_src.at[src_v.at[1]], buf1, sem1).wait()
        pltpu.sync_copy(buf1, acc.at[dst_v.at[NCHUNK - 1]], add=True)

        plsc.subcore_barrier()
        pltpu.sync_copy(acc.at[rows], out_hbm.at[c, rows])

    return aggk(feat, src3, dst3, zeros)


def _tc_matmul1(x, W1, block):
    N, Din = x.shape
    Dh = W1.shape[1]
    grid = N // block

    def body(x_ref, w_ref, h_ref):
        h_ref[...] = jnp.dot(x_ref[...], w_ref[...],
                             preferred_element_type=jnp.float32)

    return pl.pallas_call(
        body,
        grid=(grid,),
        in_specs=[
            pl.BlockSpec((block, Din), lambda i: (i, 0)),
            pl.BlockSpec((Din, Dh), lambda i: (0, 0)),
        ],
        out_specs=pl.BlockSpec((block, Dh), lambda i: (i, 0)),
        out_shape=jax.ShapeDtypeStruct((N, Dh), jnp.float32),
    )(x, W1)


def _tc_layer1(H, b1, degp, block):
    N, Dh = H.shape
    grid = N // block

    def body(h_ref, b_ref, deg_ref, hs_ref, s1_ref, dinv_ref):
        d = deg_ref[...]
        deg = d[0, :, 0:1] + d[1, :, 0:1] + 1.0
        dinv = lax.rsqrt(deg)
        H = h_ref[...]
        hs_ref[...] = H * dinv
        s1_ref[...] = b_ref[...] + H * (dinv * dinv)
        dinv_ref[...] = dinv

    return pl.pallas_call(
        body,
        grid=(grid,),
        in_specs=[
            pl.BlockSpec((block, Dh), lambda i: (i, 0)),
            pl.BlockSpec((1, Dh), lambda i: (0, 0)),
            pl.BlockSpec((NC, block, DW), lambda i: (0, i, 0)),
        ],
        out_specs=[
            pl.BlockSpec((block, Dh), lambda i: (i, 0)),
            pl.BlockSpec((block, Dh), lambda i: (i, 0)),
            pl.BlockSpec((block, 1), lambda i: (i, 0)),
        ],
        out_shape=[
            jax.ShapeDtypeStruct((N, Dh), jnp.float32),
            jax.ShapeDtypeStruct((N, Dh), jnp.float32),
            jax.ShapeDtypeStruct((N, 1), jnp.float32),
        ],
    )(H, b1, degp)


def _tc_layer2(U, dinv, S1, W2, b2, block):
    N, Dh = S1.shape
    Do = W2.shape[1]
    grid = N // block

    def body(u_ref, dinv_ref, s1_ref, w_ref, b_ref, gs_ref, s2_ref):
        u = u_ref[...]
        dinv = dinv_ref[...]
        h = jnp.maximum((u[0] + u[1]) * dinv + s1_ref[...], 0.0)
        G = jnp.dot(h, w_ref[...], preferred_element_type=jnp.float32)
        gs_ref[...] = G * dinv
        s2_ref[...] = b_ref[...] + G * (dinv * dinv)

    return pl.pallas_call(
        body,
        grid=(grid,),
        in_specs=[
            pl.BlockSpec((NC, block, Dh), lambda i: (0, i, 0)),
            pl.BlockSpec((block, 1), lambda i: (i, 0)),
            pl.BlockSpec((block, Dh), lambda i: (i, 0)),
            pl.BlockSpec((Dh, Do), lambda i: (0, 0)),
            pl.BlockSpec((1, Do), lambda i: (0, 0)),
        ],
        out_specs=[
            pl.BlockSpec((block, Do), lambda i: (i, 0)),
            pl.BlockSpec((block, Do), lambda i: (i, 0)),
        ],
        out_shape=[
            jax.ShapeDtypeStruct((N, Do), jnp.float32),
            jax.ShapeDtypeStruct((N, Do), jnp.float32),
        ],
    )(U, dinv, S1, W2, b2)


def _tc_finish(V, dinv, S2, block):
    N, Do = S2.shape
    grid = N // block

    def body(v_ref, dinv_ref, s2_ref, z_ref):
        v = v_ref[...]
        z_ref[...] = (v[0] + v[1]) * dinv_ref[...] + s2_ref[...]

    return pl.pallas_call(
        body,
        grid=(grid,),
        in_specs=[
            pl.BlockSpec((NC, block, Do), lambda i: (0, i, 0)),
            pl.BlockSpec((block, 1), lambda i: (i, 0)),
            pl.BlockSpec((block, Do), lambda i: (i, 0)),
        ],
        out_specs=pl.BlockSpec((block, Do), lambda i: (i, 0)),
        out_shape=jax.ShapeDtypeStruct((N, Do), jnp.float32),
    )(V, dinv, S2)


def kernel(x, edge_index, W1, b1, W2, b2, W3, b3):
    del W3, b3
    N, Din = x.shape
    E = edge_index.shape[1]
    Dh = W1.shape[1]
    Do = W2.shape[1]
    assert E % NW == 0 and N % NS == 0
    EPW = E // NW
    K = 100
    assert EPW % K == 0
    NCHUNK = EPW // K

    ei = edge_index.astype(jnp.int32)
    src3 = ei[0].reshape(NW, NCHUNK, K)
    dst3 = ei[1].reshape(NW, NCHUNK, K)
    NP = ((N + 8 * NS - 1) // (8 * NS)) * (8 * NS)
    ones = jnp.ones((K, DW), jnp.float32)
    zeros_h = jnp.zeros((NP, Dh), jnp.float32)
    zeros_o = jnp.zeros((NP, Do), jnp.float32)
    zeros_d = jnp.zeros((NP, DW), jnp.float32)

    block = 5000
    degp = _sc_degree(dst3, ones, zeros_d)
    H = _tc_matmul1(x, W1, block)
    Hs, S1, dinv = _tc_layer1(H, b1.reshape(1, Dh), degp, block)
    U = _sc_aggregate(Hs, src3, dst3, zeros_h)
    Gs, S2 = _tc_layer2(U, dinv, S1, W2, b2.reshape(1, Do), block)
    V = _sc_aggregate(Gs, src3, dst3, zeros_o)
    return _tc_finish(V, dinv, S2, block)
